# baseline (speedup 1.0000x reference)
import functools
import math

import jax
import jax.numpy as jnp
import numpy as np
from jax.experimental import pallas as pl
from jax.experimental.pallas import tpu as pltpu

_B, _L, _H, _D = 4, 2048, 16, 64
_FACTOR = 5
_U = min(_FACTOR * int(math.ceil(math.log(_L))), _L)

_HIGH = jax.lax.Precision.HIGHEST


def _threefry2x32(k1, k2, x0, x1):
    rot0 = (13, 15, 26, 6)
    rot1 = (17, 29, 16, 24)
    k1 = np.uint32(k1)
    k2 = np.uint32(k2)
    ks = [k1, k2, np.uint32(k1 ^ k2 ^ np.uint32(0x1BD11BDA))]
    x0 = (x0 + ks[0]).astype(np.uint32)
    x1 = (x1 + ks[1]).astype(np.uint32)

    def rnd(a, b, r):
        a = (a + b).astype(np.uint32)
        b = ((b << np.uint32(r)) | (b >> np.uint32(32 - r))).astype(np.uint32)
        return a, a ^ b

    for i, rs in enumerate([rot0, rot1, rot0, rot1, rot0]):
        for r in rs:
            x0, x1 = rnd(x0, x1, r)
        x0 = (x0 + ks[(i + 1) % 3]).astype(np.uint32)
        x1 = (x1 + ks[(i + 2) % 3] + np.uint32(i + 1)).astype(np.uint32)
    return x0, x1


def _sample_indices() -> np.ndarray:
    b1, b2 = _threefry2x32(np.uint32(0), np.uint32(42),
                           np.zeros(2, np.uint32),
                           np.arange(2, dtype=np.uint32))
    sk1, sk2 = b1[1], b2[1]
    n = _L * _U
    y0, y1 = _threefry2x32(sk1, sk2, np.zeros(n, np.uint32),
                           np.arange(n, dtype=np.uint32))
    bits = y0 ^ y1
    return (bits % np.uint32(_L)).astype(np.int32).reshape(_L, _U)


def _count_matrix() -> np.ndarray:
    idx = _sample_indices()
    c = np.zeros((_L, _L), np.int8)
    np.add.at(c, (np.arange(_L)[:, None], idx), 1)
    return c


import ml_dtypes

_COUNTS = _count_matrix()
_COUNTS_BF16 = _COUNTS.astype(np.float32).astype(ml_dtypes.bfloat16)
_NEGMASK_BF16 = np.where(_COUNTS > 0, 0.0, -1e30).astype(ml_dtypes.bfloat16)


def _m_kernel(c_ref, nm_ref, q_ref, k_ref, m_ref, *, tl):
    lt = pl.program_id(2)
    q = q_ref[0, 0, :, :].astype(jnp.bfloat16)
    k = k_ref[0, 0, :, :].astype(jnp.bfloat16)
    s = jax.lax.dot_general(
        q, k, (((1,), (1,)), ((), ())),
        preferred_element_type=jnp.float32)
    cnt = c_ref[pl.ds(lt * tl, tl), :].astype(jnp.float32)
    neg = nm_ref[pl.ds(lt * tl, tl), :].astype(jnp.float32)
    m_max = jnp.max(s + neg, axis=1)
    m_sum = jnp.sum(s * cnt, axis=1)
    m_ref[0, 0, :] = m_max - m_sum * (1.0 / _L)


def _topk_kernel(m_ref, out_ref):
    rows = _B * _H
    m = m_ref[...]
    iota_l = jax.lax.broadcasted_iota(jnp.int32, (rows, _L), 1)
    iota_u = jax.lax.broadcasted_iota(jnp.int32, (rows, 128), 1)

    def body(i, carry):
        cur, acc = carry
        vmax = jnp.max(cur, axis=1, keepdims=True)
        cand = jnp.where(cur == vmax, iota_l, _L)
        amin = jnp.min(cand, axis=1, keepdims=True)
        acc = jnp.where(iota_u == i, amin, acc)
        cur = jnp.where(iota_l == amin, -jnp.inf, cur)
        return cur, acc

    _, acc = jax.lax.fori_loop(
        0, _U, body, (m, jnp.zeros((rows, 128), jnp.int32)))
    out_ref[...] = acc


def _attn_kernel(mtop_ref, q_ref, k_ref, v_ref, out_ref, qr_ref):
    b = pl.program_id(0)
    h = pl.program_id(1)
    bh = b * _H + h
    k = k_ref[0, 0, :, :]
    v = v_ref[0, 0, :, :]

    for i in range(_U):
        qr_ref[i, :] = q_ref[0, 0, mtop_ref[bh, i], :]
    qr = qr_ref[...]

    s = jax.lax.dot_general(
        qr.astype(jnp.bfloat16), k.astype(jnp.bfloat16),
        (((1,), (1,)), ((), ())),
        preferred_element_type=jnp.float32)
    s = s * (1.0 / math.sqrt(_D))
    smax = jnp.max(s, axis=1, keepdims=True)
    e = jnp.exp(s - smax)
    p = e / jnp.sum(e, axis=1, keepdims=True)
    upd = jax.lax.dot_general(
        p.astype(jnp.bfloat16), v.astype(jnp.bfloat16),
        (((1,), (0,)), ((), ())),
        preferred_element_type=jnp.float32)

    vmean = jnp.mean(v, axis=0, keepdims=True)
    out_ref[0, 0, :, :] = jnp.broadcast_to(vmean, (_L, _D))
    for i in range(_U):
        out_ref[0, 0, pl.ds(mtop_ref[bh, i], 1), :] = upd[i:i + 1, :]


def kernel(queries, keys, values):
    B, L, H, D = queries.shape
    cnt = jnp.asarray(_COUNTS_BF16)
    neg = jnp.asarray(_NEGMASK_BF16)
    qt = jnp.transpose(queries, (0, 2, 1, 3))
    kt = jnp.transpose(keys, (0, 2, 1, 3))
    vt = jnp.transpose(values, (0, 2, 1, 3))

    tl = 1024
    m = pl.pallas_call(
        functools.partial(_m_kernel, tl=tl),
        grid=(B, H, L // tl),
        in_specs=[
            pl.BlockSpec((L, L), lambda b, h, lt: (0, 0)),
            pl.BlockSpec((L, L), lambda b, h, lt: (0, 0)),
            pl.BlockSpec((1, 1, tl, D), lambda b, h, lt: (b, h, lt, 0)),
            pl.BlockSpec((1, 1, L, D), lambda b, h, lt: (b, h, 0, 0)),
        ],
        out_specs=pl.BlockSpec((1, 1, tl), lambda b, h, lt: (b * H + h, 0, lt)),
        out_shape=jax.ShapeDtypeStruct((B * H, 1, L), jnp.float32),
    )(cnt, neg, qt, kt)

    m2 = m.reshape(B * H, L)
    mtop = pl.pallas_call(
        _topk_kernel,
        in_specs=[pl.BlockSpec((B * H, L), lambda: (0, 0))],
        out_specs=pl.BlockSpec((B * H, 128), lambda: (0, 0)),
        out_shape=jax.ShapeDtypeStruct((B * H, 128), jnp.int32),
    )(m2)

    out = pl.pallas_call(
        _attn_kernel,
        grid_spec=pltpu.PrefetchScalarGridSpec(
            num_scalar_prefetch=1,
            grid=(B, H),
            in_specs=[
                pl.BlockSpec((1, 1, L, D), lambda b, h, mt: (b, h, 0, 0)),
                pl.BlockSpec((1, 1, L, D), lambda b, h, mt: (b, h, 0, 0)),
                pl.BlockSpec((1, 1, L, D), lambda b, h, mt: (b, h, 0, 0)),
            ],
            out_specs=pl.BlockSpec((1, 1, L, D), lambda b, h, mt: (b, h, 0, 0)),
            scratch_shapes=[pltpu.VMEM((_U, D), jnp.float32)],
        ),
        out_shape=jax.ShapeDtypeStruct((B, H, L, D), jnp.float32),
    )(mtop, qt, kt, vt)
    return jnp.transpose(out, (0, 2, 1, 3))


# speedup vs baseline: 1.0195x; 1.0195x over previous
import functools
import math

import jax
import jax.numpy as jnp
import numpy as np
from jax.experimental import pallas as pl
from jax.experimental.pallas import tpu as pltpu

_B, _L, _H, _D = 4, 2048, 16, 64
_FACTOR = 5
_U = min(_FACTOR * int(math.ceil(math.log(_L))), _L)

_HIGH = jax.lax.Precision.HIGHEST


def _threefry2x32(k1, k2, x0, x1):
    rot0 = (13, 15, 26, 6)
    rot1 = (17, 29, 16, 24)
    k1 = np.uint32(k1)
    k2 = np.uint32(k2)
    ks = [k1, k2, np.uint32(k1 ^ k2 ^ np.uint32(0x1BD11BDA))]
    x0 = (x0 + ks[0]).astype(np.uint32)
    x1 = (x1 + ks[1]).astype(np.uint32)

    def rnd(a, b, r):
        a = (a + b).astype(np.uint32)
        b = ((b << np.uint32(r)) | (b >> np.uint32(32 - r))).astype(np.uint32)
        return a, a ^ b

    for i, rs in enumerate([rot0, rot1, rot0, rot1, rot0]):
        for r in rs:
            x0, x1 = rnd(x0, x1, r)
        x0 = (x0 + ks[(i + 1) % 3]).astype(np.uint32)
        x1 = (x1 + ks[(i + 2) % 3] + np.uint32(i + 1)).astype(np.uint32)
    return x0, x1


def _sample_indices() -> np.ndarray:
    b1, b2 = _threefry2x32(np.uint32(0), np.uint32(42),
                           np.zeros(2, np.uint32),
                           np.arange(2, dtype=np.uint32))
    sk1, sk2 = b1[1], b2[1]
    n = _L * _U
    y0, y1 = _threefry2x32(sk1, sk2, np.zeros(n, np.uint32),
                           np.arange(n, dtype=np.uint32))
    bits = y0 ^ y1
    return (bits % np.uint32(_L)).astype(np.int32).reshape(_L, _U)


def _count_matrix() -> np.ndarray:
    idx = _sample_indices()
    c = np.zeros((_L, _L), np.int8)
    np.add.at(c, (np.arange(_L)[:, None], idx), 1)
    return c


import ml_dtypes

_COUNTS = _count_matrix()
_COUNTS_BF16 = _COUNTS.astype(np.float32).astype(ml_dtypes.bfloat16)
_NEGMASK_BF16 = np.where(_COUNTS > 0, 0.0, -1e30).astype(ml_dtypes.bfloat16)


def _m_kernel(c_ref, nm_ref, q_ref, k_ref, m_ref, *, tl):
    lt = pl.program_id(2)
    q = q_ref[0, 0, :, :].astype(jnp.bfloat16)
    k = k_ref[0, 0, :, :].astype(jnp.bfloat16)
    s = jax.lax.dot_general(
        q, k, (((1,), (1,)), ((), ())),
        preferred_element_type=jnp.float32)
    cnt = c_ref[pl.ds(lt * tl, tl), :].astype(jnp.float32)
    neg = nm_ref[pl.ds(lt * tl, tl), :].astype(jnp.float32)
    m_max = jnp.max(s + neg, axis=1)
    m_sum = jnp.sum(s * cnt, axis=1)
    m_ref[0, 0, :] = m_max - m_sum * (1.0 / _L)


def _topk_kernel(m_ref, out_ref):
    rows = _B * _H
    m = m_ref[...]
    iota_l = jax.lax.broadcasted_iota(jnp.int32, (rows, _L), 1)
    iota_u = jax.lax.broadcasted_iota(jnp.int32, (rows, 128), 1)

    def body(i, carry):
        cur, acc = carry
        vmax = jnp.max(cur, axis=1, keepdims=True)
        cand = jnp.where(cur == vmax, iota_l, _L)
        amin = jnp.min(cand, axis=1, keepdims=True)
        acc = jnp.where(iota_u == i, amin, acc)
        cur = jnp.where(iota_l == amin, -jnp.inf, cur)
        return cur, acc

    _, acc = jax.lax.fori_loop(
        0, _U, body, (m, jnp.zeros((rows, 128), jnp.int32)))
    out_ref[...] = acc


def _attn_kernel(mtop_ref, q_ref, k_ref, v_ref, out_ref, qr_ref):
    b = pl.program_id(0)
    h = pl.program_id(1)
    bh = b * _H + h
    k = k_ref[0, 0, :, :]
    v = v_ref[0, 0, :, :]

    for i in range(_U):
        qr_ref[i, :] = q_ref[0, 0, mtop_ref[bh, i], :]
    qr = qr_ref[...]

    s = jax.lax.dot_general(
        qr.astype(jnp.bfloat16), k.astype(jnp.bfloat16),
        (((1,), (1,)), ((), ())),
        preferred_element_type=jnp.float32)
    s = s * (1.0 / math.sqrt(_D))
    smax = jnp.max(s, axis=1, keepdims=True)
    e = jnp.exp(s - smax)
    p = e / jnp.sum(e, axis=1, keepdims=True)
    upd = jax.lax.dot_general(
        p.astype(jnp.bfloat16), v.astype(jnp.bfloat16),
        (((1,), (0,)), ((), ())),
        preferred_element_type=jnp.float32)

    vmean = jnp.mean(v, axis=0, keepdims=True)
    out_ref[0, 0, :, :] = jnp.broadcast_to(vmean, (_L, _D))
    for i in range(_U):
        out_ref[0, 0, pl.ds(mtop_ref[bh, i], 1), :] = upd[i:i + 1, :]


def kernel(queries, keys, values):
    B, L, H, D = queries.shape
    cnt = jnp.asarray(_COUNTS_BF16)
    neg = jnp.asarray(_NEGMASK_BF16)
    qt = jnp.transpose(queries, (0, 2, 1, 3))
    kt = jnp.transpose(keys, (0, 2, 1, 3))
    vt = jnp.transpose(values, (0, 2, 1, 3))

    tl = 512
    m = pl.pallas_call(
        functools.partial(_m_kernel, tl=tl),
        grid=(B, H, L // tl),
        in_specs=[
            pl.BlockSpec((L, L), lambda b, h, lt: (0, 0)),
            pl.BlockSpec((L, L), lambda b, h, lt: (0, 0)),
            pl.BlockSpec((1, 1, tl, D), lambda b, h, lt: (b, h, lt, 0)),
            pl.BlockSpec((1, 1, L, D), lambda b, h, lt: (b, h, 0, 0)),
        ],
        out_specs=pl.BlockSpec((1, 1, tl), lambda b, h, lt: (b * H + h, 0, lt)),
        out_shape=jax.ShapeDtypeStruct((B * H, 1, L), jnp.float32),
    )(cnt, neg, qt, kt)

    m2 = m.reshape(B * H, L)
    mtop = pl.pallas_call(
        _topk_kernel,
        in_specs=[pl.BlockSpec((B * H, L), lambda: (0, 0))],
        out_specs=pl.BlockSpec((B * H, 128), lambda: (0, 0)),
        out_shape=jax.ShapeDtypeStruct((B * H, 128), jnp.int32),
    )(m2)

    out = pl.pallas_call(
        _attn_kernel,
        grid_spec=pltpu.PrefetchScalarGridSpec(
            num_scalar_prefetch=1,
            grid=(B, H),
            in_specs=[
                pl.BlockSpec((1, 1, L, D), lambda b, h, mt: (b, h, 0, 0)),
                pl.BlockSpec((1, 1, L, D), lambda b, h, mt: (b, h, 0, 0)),
                pl.BlockSpec((1, 1, L, D), lambda b, h, mt: (b, h, 0, 0)),
            ],
            out_specs=pl.BlockSpec((1, 1, L, D), lambda b, h, mt: (b, h, 0, 0)),
            scratch_shapes=[pltpu.VMEM((_U, D), jnp.float32)],
        ),
        out_shape=jax.ShapeDtypeStruct((B, H, L, D), jnp.float32),
    )(mtop, qt, kt, vt)
    return jnp.transpose(out, (0, 2, 1, 3))


# speedup vs baseline: 1.1361x; 1.1143x over previous
import functools
import math

import jax
import jax.numpy as jnp
import numpy as np
from jax.experimental import pallas as pl
from jax.experimental.pallas import tpu as pltpu

_B, _L, _H, _D = 4, 2048, 16, 64
_FACTOR = 5
_U = min(_FACTOR * int(math.ceil(math.log(_L))), _L)

_HIGH = jax.lax.Precision.HIGHEST


def _threefry2x32(k1, k2, x0, x1):
    rot0 = (13, 15, 26, 6)
    rot1 = (17, 29, 16, 24)
    k1 = np.uint32(k1)
    k2 = np.uint32(k2)
    ks = [k1, k2, np.uint32(k1 ^ k2 ^ np.uint32(0x1BD11BDA))]
    x0 = (x0 + ks[0]).astype(np.uint32)
    x1 = (x1 + ks[1]).astype(np.uint32)

    def rnd(a, b, r):
        a = (a + b).astype(np.uint32)
        b = ((b << np.uint32(r)) | (b >> np.uint32(32 - r))).astype(np.uint32)
        return a, a ^ b

    for i, rs in enumerate([rot0, rot1, rot0, rot1, rot0]):
        for r in rs:
            x0, x1 = rnd(x0, x1, r)
        x0 = (x0 + ks[(i + 1) % 3]).astype(np.uint32)
        x1 = (x1 + ks[(i + 2) % 3] + np.uint32(i + 1)).astype(np.uint32)
    return x0, x1


def _sample_indices() -> np.ndarray:
    b1, b2 = _threefry2x32(np.uint32(0), np.uint32(42),
                           np.zeros(2, np.uint32),
                           np.arange(2, dtype=np.uint32))
    sk1, sk2 = b1[1], b2[1]
    n = _L * _U
    y0, y1 = _threefry2x32(sk1, sk2, np.zeros(n, np.uint32),
                           np.arange(n, dtype=np.uint32))
    bits = y0 ^ y1
    return (bits % np.uint32(_L)).astype(np.int32).reshape(_L, _U)


def _count_matrix() -> np.ndarray:
    idx = _sample_indices()
    c = np.zeros((_L, _L), np.int8)
    np.add.at(c, (np.arange(_L)[:, None], idx), 1)
    return c


_COUNTS = _count_matrix()
_COUNTS_F32 = _COUNTS.astype(np.float32)
_NEGMASK_F32 = np.where(_COUNTS > 0, 0.0, -1e30).astype(np.float32)


def _m_kernel(c_ref, nm_ref, q_ref, k_ref, m_ref, *, tl, kc):
    lt = pl.program_id(2)
    q = q_ref[0, 0, :, :].astype(jnp.bfloat16)
    m_max = None
    m_sum = None
    for c in range(_L // kc):
        k = k_ref[0, 0, c * kc:(c + 1) * kc, :].astype(jnp.bfloat16)
        s = jax.lax.dot_general(
            q, k, (((1,), (1,)), ((), ())),
            preferred_element_type=jnp.float32)
        cnt = c_ref[pl.ds(lt * tl, tl), c * kc:(c + 1) * kc]
        neg = nm_ref[pl.ds(lt * tl, tl), c * kc:(c + 1) * kc]
        cmax = jnp.max(s + neg, axis=1)
        csum = jnp.sum(s * cnt, axis=1)
        m_max = cmax if m_max is None else jnp.maximum(m_max, cmax)
        m_sum = csum if m_sum is None else m_sum + csum
    m_ref[0, 0, :] = m_max - m_sum * (1.0 / _L)


def _topk_kernel(m_ref, out_ref):
    rows = _B * _H
    m = m_ref[...]
    iota_l = jax.lax.broadcasted_iota(jnp.int32, (rows, _L), 1)
    iota_u = jax.lax.broadcasted_iota(jnp.int32, (rows, 128), 1)

    def body(i, carry):
        cur, acc = carry
        vmax = jnp.max(cur, axis=1, keepdims=True)
        cand = jnp.where(cur == vmax, iota_l, _L)
        amin = jnp.min(cand, axis=1, keepdims=True)
        acc = jnp.where(iota_u == i, amin, acc)
        cur = jnp.where(iota_l == amin, -jnp.inf, cur)
        return cur, acc

    _, acc = jax.lax.fori_loop(
        0, _U, body, (m, jnp.zeros((rows, 128), jnp.int32)))
    out_ref[...] = acc


def _attn_kernel(mtop_ref, q_ref, k_ref, v_ref, out_ref, qr_ref):
    b = pl.program_id(0)
    h = pl.program_id(1)
    bh = b * _H + h
    k = k_ref[0, 0, :, :]
    v = v_ref[0, 0, :, :]

    for i in range(_U):
        qr_ref[i, :] = q_ref[0, 0, mtop_ref[bh, i], :]
    qr = qr_ref[...]

    s = jax.lax.dot_general(
        qr.astype(jnp.bfloat16), k.astype(jnp.bfloat16),
        (((1,), (1,)), ((), ())),
        preferred_element_type=jnp.float32)
    s = s * (1.0 / math.sqrt(_D))
    smax = jnp.max(s, axis=1, keepdims=True)
    e = jnp.exp(s - smax)
    p = e / jnp.sum(e, axis=1, keepdims=True)
    upd = jax.lax.dot_general(
        p.astype(jnp.bfloat16), v.astype(jnp.bfloat16),
        (((1,), (0,)), ((), ())),
        preferred_element_type=jnp.float32)

    vmean = jnp.mean(v, axis=0, keepdims=True)
    out_ref[0, 0, :, :] = jnp.broadcast_to(vmean, (_L, _D))
    for i in range(_U):
        out_ref[0, 0, pl.ds(mtop_ref[bh, i], 1), :] = upd[i:i + 1, :]


def kernel(queries, keys, values):
    B, L, H, D = queries.shape
    cnt = jnp.asarray(_COUNTS_F32)
    neg = jnp.asarray(_NEGMASK_F32)
    qt = jnp.transpose(queries, (0, 2, 1, 3))
    kt = jnp.transpose(keys, (0, 2, 1, 3))
    vt = jnp.transpose(values, (0, 2, 1, 3))

    tl = 512
    m = pl.pallas_call(
        functools.partial(_m_kernel, tl=tl, kc=512),
        grid=(B, H, L // tl),
        in_specs=[
            pl.BlockSpec((L, L), lambda b, h, lt: (0, 0)),
            pl.BlockSpec((L, L), lambda b, h, lt: (0, 0)),
            pl.BlockSpec((1, 1, tl, D), lambda b, h, lt: (b, h, lt, 0)),
            pl.BlockSpec((1, 1, L, D), lambda b, h, lt: (b, h, 0, 0)),
        ],
        out_specs=pl.BlockSpec((1, 1, tl), lambda b, h, lt: (b * H + h, 0, lt)),
        out_shape=jax.ShapeDtypeStruct((B * H, 1, L), jnp.float32),
    )(cnt, neg, qt, kt)

    m2 = m.reshape(B * H, L)
    mtop = pl.pallas_call(
        _topk_kernel,
        in_specs=[pl.BlockSpec((B * H, L), lambda: (0, 0))],
        out_specs=pl.BlockSpec((B * H, 128), lambda: (0, 0)),
        out_shape=jax.ShapeDtypeStruct((B * H, 128), jnp.int32),
    )(m2)

    out = pl.pallas_call(
        _attn_kernel,
        grid_spec=pltpu.PrefetchScalarGridSpec(
            num_scalar_prefetch=1,
            grid=(B, H),
            in_specs=[
                pl.BlockSpec((1, 1, L, D), lambda b, h, mt: (b, h, 0, 0)),
                pl.BlockSpec((1, 1, L, D), lambda b, h, mt: (b, h, 0, 0)),
                pl.BlockSpec((1, 1, L, D), lambda b, h, mt: (b, h, 0, 0)),
            ],
            out_specs=pl.BlockSpec((1, 1, L, D), lambda b, h, mt: (b, h, 0, 0)),
            scratch_shapes=[pltpu.VMEM((_U, D), jnp.float32)],
        ),
        out_shape=jax.ShapeDtypeStruct((B, H, L, D), jnp.float32),
    )(mtop, qt, kt, vt)
    return jnp.transpose(out, (0, 2, 1, 3))
